# baseline (speedup 1.0000x reference)
import functools

import jax
import jax.numpy as jnp
from jax import lax
from jax.experimental import pallas as pl
from jax.experimental.pallas import tpu as pltpu
from jax.experimental.pallas import tpu_sc as plsc

MASK = 4194304 - 1
P1 = 19349663
P2 = 83492791
L = 16


def _mesh(NC, NS):
    return plsc.VectorSubcoreMesh(
        core_axis_name="c", subcore_axis_name="s",
        num_cores=NC, num_subcores=NS)


@functools.cache
def _make_k1(V, D, NC, NS):
    NW = NC * NS
    CW = 2048
    cols_w = V // NW
    n_sub = cols_w // CW
    OUT_CH = CW * D // 128

    @functools.partial(
        pl.kernel,
        out_type=jax.ShapeDtypeStruct((V * D // 128, 128), jnp.float32),
        mesh=_mesh(NC, NS),
        scratch_types=[
            pltpu.VMEM((D, CW), jnp.float32),
            pltpu.VMEM((OUT_CH, 128), jnp.float32),
        ],
        compiler_params=pltpu.CompilerParams(use_tc_tiling_on_sc=True, needs_layout_passes=False),
    )
    def k1(tab_hbm, out_hbm, vin, vout):
        wid = lax.axis_index("s") * NC + lax.axis_index("c")
        iota = lax.iota(jnp.int32, L)

        def sub_body(s, carry):
            col0 = pl.multiple_of(wid * cols_w + s * CW, 128)
            pltpu.sync_copy(tab_hbm.at[:, pl.ds(col0, CW)], vin)

            def tr_body(j, carry2):
                g = plsc.load_gather(vin, [iota, jnp.full((L,), j, jnp.int32)])
                vout[j // 8, pl.ds((j % 8) * L, L)] = g
                return carry2

            lax.fori_loop(jnp.int32(0), jnp.int32(CW), tr_body, 0)
            orow0 = pl.multiple_of((wid * cols_w + s * CW) * D // 128, 8)
            pltpu.sync_copy(vout, out_hbm.at[pl.ds(orow0, OUT_CH)])
            return carry

        lax.fori_loop(jnp.int32(0), jnp.int32(n_sub), sub_body, 0)

    return k1


@functools.cache
def _make_k2(B, V, D, NC, NS):
    NW = NC * NS
    b_w = B // NW
    CH = 2048
    n_sub = b_w // CH
    TC_CH = CH // 128

    @functools.partial(
        pl.kernel,
        out_type=jax.ShapeDtypeStruct((D // 8, B // 128, 8, 128), jnp.float32),
        mesh=_mesh(NC, NS),
        scratch_types=[
            pltpu.VMEM((CH,), jnp.int32),
            pltpu.VMEM((CH,), jnp.int32),
            pltpu.VMEM((CH,), jnp.int32),
            pltpu.VMEM((CH,), jnp.int32),
            pltpu.VMEM((CH, D), jnp.float32),
            pltpu.VMEM((D // 8, TC_CH, 8, 128), jnp.float32),
            pltpu.SemaphoreType.DMA,
        ],
        compiler_params=pltpu.CompilerParams(use_tc_tiling_on_sc=False, needs_layout_passes=False),
    )
    def k2(i0_hbm, i1_hbm, i2_hbm, table_hbm, out_hbm,
           i0_v, i1_v, i2_v, h_v, rows_v, vout, sem):
        wid = lax.axis_index("s") * NC + lax.axis_index("c")
        base_w = wid * b_w
        iota = lax.iota(jnp.int32, L)

        def sub_body(s, carry):
            base = base_w + s * CH
            pltpu.sync_copy(i0_hbm.at[pl.ds(base, CH)], i0_v)
            pltpu.sync_copy(i1_hbm.at[pl.ds(base, CH)], i1_v)
            pltpu.sync_copy(i2_hbm.at[pl.ds(base, CH)], i2_v)

            def hash_body(j, carry2):
                a = i0_v[pl.ds(j * L, L)]
                b = i1_v[pl.ds(j * L, L)]
                c = i2_v[pl.ds(j * L, L)]
                h_v[pl.ds(j * L, L)] = (a ^ (b * P1) ^ (c * P2)) & MASK
                return carry2

            lax.fori_loop(jnp.int32(0), jnp.int32(CH // L), hash_body, 0)
            pltpu.async_copy(table_hbm.at[h_v], rows_v, sem).wait()

            def tr_body(t, carry2):
                f = t // (CH // L)
                b16 = t % (CH // L)
                g = plsc.load_gather(
                    rows_v, [b16 * L + iota, jnp.full((L,), f, jnp.int32)])
                vout[f // 8, b16 // 8, f % 8, pl.ds((b16 % 8) * L, L)] = g
                return carry2

            lax.fori_loop(jnp.int32(0), jnp.int32(D * (CH // L)), tr_body, 0)

            tc0 = base // 128
            z, o = jnp.int32(0), jnp.int32(1)
            pltpu.sync_copy(vout.at[z], out_hbm.at[z].at[pl.ds(tc0, TC_CH)])
            pltpu.sync_copy(vout.at[o], out_hbm.at[o].at[pl.ds(tc0, TC_CH)])
            return carry

        lax.fori_loop(jnp.int32(0), jnp.int32(n_sub), sub_body, 0)

    return k2


def kernel(index, hash_table):
    B = index.shape[0]
    V, D = hash_table.shape
    try:
        info = plsc.get_sparse_core_info()
        NC, NS = info.num_cores, info.num_subcores
    except Exception:
        NC, NS = 2, 16
    idx32 = index.astype(jnp.int32)
    table_lin = _make_k1(V, D, NC, NS)(hash_table.T)
    out4d = _make_k2(B, V, D, NC, NS)(
        idx32[:, 0], idx32[:, 1], idx32[:, 2],
        table_lin.reshape(V, D))
    return jnp.transpose(out4d, (1, 3, 0, 2)).reshape(B, D)


# speedup vs baseline: 1.0609x; 1.0609x over previous
import functools

import jax
import jax.numpy as jnp
from jax import lax
from jax.experimental import pallas as pl
from jax.experimental.pallas import tpu as pltpu
from jax.experimental.pallas import tpu_sc as plsc

MASK = 4194304 - 1
P1 = 19349663
P2 = 83492791
L = 16


def _mesh(NC, NS):
    return plsc.VectorSubcoreMesh(
        core_axis_name="c", subcore_axis_name="s",
        num_cores=NC, num_subcores=NS)


@functools.cache
def _make_k1(V, D, NC, NS):
    NW = NC * NS
    CW = 2048
    CWP = CW + 1
    cols_w = V // NW
    n_sub = cols_w // CW
    OUT_CH = CW * D // 128
    UNR = 16

    @functools.partial(
        pl.kernel,
        out_type=jax.ShapeDtypeStruct((V * D // 128, 128), jnp.float32),
        mesh=_mesh(NC, NS),
        scratch_types=[
            pltpu.VMEM((D, CWP), jnp.float32),
            pltpu.VMEM((OUT_CH, 128), jnp.float32),
        ],
        compiler_params=pltpu.CompilerParams(
            use_tc_tiling_on_sc=True, needs_layout_passes=False),
    )
    def k1(tab_hbm, out_hbm, vin, vout):
        wid = lax.axis_index("s") * NC + lax.axis_index("c")
        iota = lax.iota(jnp.int32, L)

        def sub_body(s, carry):
            col0 = pl.multiple_of(wid * cols_w + s * CW, 128)
            pltpu.sync_copy(tab_hbm.at[:, pl.ds(col0, CW)],
                            vin.at[:, pl.ds(0, CW)])

            def tr_body(g, carry2):
                j0 = g * UNR
                for u in range(UNR):
                    vec = plsc.load_gather(
                        vin, [iota, jnp.full((L,), j0 + u, jnp.int32)])
                    vout[2 * g + u // 8, pl.ds((u % 8) * L, L)] = vec
                return carry2

            lax.fori_loop(jnp.int32(0), jnp.int32(CW // UNR), tr_body, 0)
            orow0 = pl.multiple_of((wid * cols_w + s * CW) * D // 128, 8)
            pltpu.sync_copy(vout, out_hbm.at[pl.ds(orow0, OUT_CH)])
            return carry

        lax.fori_loop(jnp.int32(0), jnp.int32(n_sub), sub_body, 0)

    return k1


@functools.cache
def _make_k2(B, V, D, NC, NS):
    NW = NC * NS
    b_w = B // NW
    CH = 2048
    n_sub = b_w // CH
    TC_CH = CH // 128
    VR = TC_CH * 8
    UNR = 8

    @functools.partial(
        pl.kernel,
        out_type=jax.ShapeDtypeStruct((D // 8, B // 128 * 8, 128),
                                      jnp.float32),
        mesh=_mesh(NC, NS),
        scratch_types=[
            pltpu.VMEM((CH,), jnp.int32),
            pltpu.VMEM((CH,), jnp.int32),
            pltpu.VMEM((CH,), jnp.int32),
            pltpu.VMEM((CH,), jnp.int32),
            pltpu.VMEM((CH, D), jnp.float32),
            pltpu.VMEM((2 * VR, 129), jnp.float32),
            pltpu.SemaphoreType.DMA,
        ],
        compiler_params=pltpu.CompilerParams(
            use_tc_tiling_on_sc=False, needs_layout_passes=False),
    )
    def k2(i0_hbm, i1_hbm, i2_hbm, table_hbm, out_hbm,
           i0_v, i1_v, i2_v, h_v, rows_v, vout, sem):
        wid = lax.axis_index("s") * NC + lax.axis_index("c")
        base_w = wid * b_w
        iota = lax.iota(jnp.int32, L)
        row_const = (iota // 8) * VR + (iota % 8)

        def sub_body(s, carry):
            base = base_w + s * CH
            pltpu.sync_copy(i0_hbm.at[pl.ds(base, CH)], i0_v)
            pltpu.sync_copy(i1_hbm.at[pl.ds(base, CH)], i1_v)
            pltpu.sync_copy(i2_hbm.at[pl.ds(base, CH)], i2_v)

            def hash_body(j, carry2):
                a = i0_v[pl.ds(j * L, L)]
                b = i1_v[pl.ds(j * L, L)]
                c = i2_v[pl.ds(j * L, L)]
                h_v[pl.ds(j * L, L)] = (a ^ (b * P1) ^ (c * P2)) & MASK
                return carry2

            lax.fori_loop(jnp.int32(0), jnp.int32(CH // L), hash_body, 0)
            pltpu.async_copy(table_hbm.at[h_v], rows_v, sem).wait()

            def tr_body(g, carry2):
                b0 = g * UNR
                rvec = row_const + (b0 // 128) * 8
                c0 = b0 % 128
                for u in range(UNR):
                    v = rows_v[b0 + u, pl.ds(0, L)]
                    plsc.store_scatter(
                        vout, [rvec, jnp.full((L,), c0 + u, jnp.int32)], v)
                return carry2

            lax.fori_loop(jnp.int32(0), jnp.int32(CH // UNR), tr_body, 0)

            tcr0 = base // 128 * 8
            z, o = jnp.int32(0), jnp.int32(1)
            pltpu.sync_copy(vout.at[pl.ds(0, VR), pl.ds(0, 128)],
                            out_hbm.at[z].at[pl.ds(tcr0, VR)])
            pltpu.sync_copy(vout.at[pl.ds(VR, VR), pl.ds(0, 128)],
                            out_hbm.at[o].at[pl.ds(tcr0, VR)])
            return carry

        lax.fori_loop(jnp.int32(0), jnp.int32(n_sub), sub_body, 0)

    return k2


def kernel(index, hash_table):
    B = index.shape[0]
    V, D = hash_table.shape
    try:
        info = plsc.get_sparse_core_info()
        NC, NS = info.num_cores, info.num_subcores
    except Exception:
        NC, NS = 2, 16
    idx32 = index.astype(jnp.int32)
    table_lin = _make_k1(V, D, NC, NS)(hash_table.T)
    out3d = _make_k2(B, V, D, NC, NS)(
        idx32[:, 0], idx32[:, 1], idx32[:, 2],
        table_lin.reshape(V, D))
    return (out3d.reshape(D // 8, B // 128, 8, 128)
            .transpose(1, 3, 0, 2).reshape(B, D))


# speedup vs baseline: 1.8194x; 1.7150x over previous
import functools

import jax
import jax.numpy as jnp
from jax import lax
from jax.experimental import pallas as pl
from jax.experimental.pallas import tpu as pltpu
from jax.experimental.pallas import tpu_sc as plsc

MASK = 4194304 - 1
P1 = 19349663
P2 = 83492791
L = 16


def _mesh(NC, NS):
    return plsc.VectorSubcoreMesh(
        core_axis_name="c", subcore_axis_name="s",
        num_cores=NC, num_subcores=NS)


@functools.cache
def _make_k1(V, D, NC, NS):
    NW = NC * NS
    CW = 2048
    cols_w = V // NW
    n_sub = cols_w // CW
    OUT_CH = CW * D // 128
    UNR = 16
    VR2 = 136

    @functools.partial(
        pl.kernel,
        out_type=jax.ShapeDtypeStruct((V * D // 128, 128), jnp.float32),
        mesh=_mesh(NC, NS),
        scratch_types=[
            pltpu.VMEM((2 * VR2, 129), jnp.float32),
            pltpu.VMEM((OUT_CH, 128), jnp.float32),
        ],
        compiler_params=pltpu.CompilerParams(
            use_tc_tiling_on_sc=False, needs_layout_passes=False),
    )
    def k1(tab_hbm, out_hbm, vin, vout):
        wid = lax.axis_index("s") * NC + lax.axis_index("c")
        iota = lax.iota(jnp.int32, L)
        row_base = (iota // 8) * VR2 + (iota % 8)
        z, o = jnp.int32(0), jnp.int32(1)

        def sub_body(s, carry):
            col0 = wid * cols_w + s * CW
            tcb = col0 // 128
            pltpu.sync_copy(tab_hbm.at[z].at[pl.ds(tcb * 8, 128)],
                            vin.at[pl.ds(0, 128), pl.ds(0, 128)])
            pltpu.sync_copy(tab_hbm.at[o].at[pl.ds(tcb * 8, 128)],
                            vin.at[pl.ds(VR2, 128), pl.ds(0, 128)])

            def tr_body(g, carry2):
                rvec = row_base + (g // 8) * 8
                c0 = (g % 8) * L
                for u in range(UNR):
                    vec = plsc.load_gather(
                        vin, [rvec, jnp.full((L,), c0 + u, jnp.int32)])
                    vout[2 * g + u // 8, pl.ds((u % 8) * L, L)] = vec
                return carry2

            lax.fori_loop(jnp.int32(0), jnp.int32(CW // UNR), tr_body, 0)
            orow0 = (wid * cols_w + s * CW) * D // 128
            pltpu.sync_copy(vout, out_hbm.at[pl.ds(orow0, OUT_CH)])
            return carry

        lax.fori_loop(jnp.int32(0), jnp.int32(n_sub), sub_body, 0)

    return k1


@functools.cache
def _make_k2(B, V, D, NC, NS):
    NW = NC * NS
    b_w = B // NW
    CH = 2048
    n_sub = b_w // CH
    TC_CH = CH // 128
    VR = TC_CH * 8
    UNR = 8

    @functools.partial(
        pl.kernel,
        out_type=jax.ShapeDtypeStruct((D // 8, B // 128 * 8, 128),
                                      jnp.float32),
        mesh=_mesh(NC, NS),
        scratch_types=[
            pltpu.VMEM((CH,), jnp.int32),
            pltpu.VMEM((CH,), jnp.int32),
            pltpu.VMEM((CH,), jnp.int32),
            pltpu.VMEM((CH,), jnp.int32),
            pltpu.VMEM((CH, D), jnp.float32),
            pltpu.VMEM((2 * VR, 129), jnp.float32),
            pltpu.SemaphoreType.DMA,
        ],
        compiler_params=pltpu.CompilerParams(
            use_tc_tiling_on_sc=False, needs_layout_passes=False),
    )
    def k2(i0_hbm, i1_hbm, i2_hbm, table_hbm, out_hbm,
           i0_v, i1_v, i2_v, h_v, rows_v, vout, sem):
        wid = lax.axis_index("s") * NC + lax.axis_index("c")
        base_w = wid * b_w
        iota = lax.iota(jnp.int32, L)
        row_const = (iota // 8) * VR + (iota % 8)

        def sub_body(s, carry):
            base = base_w + s * CH
            pltpu.sync_copy(i0_hbm.at[pl.ds(base, CH)], i0_v)
            pltpu.sync_copy(i1_hbm.at[pl.ds(base, CH)], i1_v)
            pltpu.sync_copy(i2_hbm.at[pl.ds(base, CH)], i2_v)

            def hash_body(j, carry2):
                a = i0_v[pl.ds(j * L, L)]
                b = i1_v[pl.ds(j * L, L)]
                c = i2_v[pl.ds(j * L, L)]
                h_v[pl.ds(j * L, L)] = (a ^ (b * P1) ^ (c * P2)) & MASK
                return carry2

            lax.fori_loop(jnp.int32(0), jnp.int32(CH // L), hash_body, 0)
            pltpu.async_copy(table_hbm.at[h_v], rows_v, sem).wait()

            def tr_body(g, carry2):
                b0 = g * UNR
                rvec = row_const + (b0 // 128) * 8
                c0 = b0 % 128
                for u in range(UNR):
                    v = rows_v[b0 + u, pl.ds(0, L)]
                    plsc.store_scatter(
                        vout, [rvec, jnp.full((L,), c0 + u, jnp.int32)], v)
                return carry2

            lax.fori_loop(jnp.int32(0), jnp.int32(CH // UNR), tr_body, 0)

            tcr0 = base // 128 * 8
            z, o = jnp.int32(0), jnp.int32(1)
            pltpu.sync_copy(vout.at[pl.ds(0, VR), pl.ds(0, 128)],
                            out_hbm.at[z].at[pl.ds(tcr0, VR)])
            pltpu.sync_copy(vout.at[pl.ds(VR, VR), pl.ds(0, 128)],
                            out_hbm.at[o].at[pl.ds(tcr0, VR)])
            return carry

        lax.fori_loop(jnp.int32(0), jnp.int32(n_sub), sub_body, 0)

    return k2


def kernel(index, hash_table):
    B = index.shape[0]
    V, D = hash_table.shape
    try:
        info = plsc.get_sparse_core_info()
        NC, NS = info.num_cores, info.num_subcores
    except Exception:
        NC, NS = 2, 16
    idx32 = index.astype(jnp.int32)
    tab_bytes = (hash_table.reshape(V // 128, 128, D // 8, 8)
                 .transpose(2, 0, 3, 1).reshape(D // 8, V // 128 * 8, 128))
    table_lin = _make_k1(V, D, NC, NS)(tab_bytes)
    out3d = _make_k2(B, V, D, NC, NS)(
        idx32[:, 0], idx32[:, 1], idx32[:, 2],
        table_lin.reshape(V, D))
    return (out3d.reshape(D // 8, B // 128, 8, 128)
            .transpose(1, 3, 0, 2).reshape(B, D))


# speedup vs baseline: 1.9934x; 1.0956x over previous
import functools

import jax
import jax.numpy as jnp
from jax import lax
from jax.experimental import pallas as pl
from jax.experimental.pallas import tpu as pltpu
from jax.experimental.pallas import tpu_sc as plsc

MASK = 4194304 - 1
P1 = 19349663
P2 = 83492791
L = 16


def _mesh(NC, NS):
    return plsc.VectorSubcoreMesh(
        core_axis_name="c", subcore_axis_name="s",
        num_cores=NC, num_subcores=NS)


@functools.cache
def _make_k1(V, D, NC, NS):
    NW = NC * NS
    CW = 2048
    cols_w = V // NW
    n_sub = cols_w // CW
    OUT_CH = CW * D // 128
    UNR = 16
    VR2 = 136

    @functools.partial(
        pl.kernel,
        out_type=jax.ShapeDtypeStruct((V * D // 128, 128), jnp.float32),
        mesh=_mesh(NC, NS),
        scratch_types=[
            pltpu.VMEM((2 * VR2, 129), jnp.float32),
            pltpu.VMEM((2 * VR2, 129), jnp.float32),
            pltpu.VMEM((OUT_CH, 128), jnp.float32),
            pltpu.SemaphoreType.DMA,
            pltpu.SemaphoreType.DMA,
        ],
        compiler_params=pltpu.CompilerParams(
            use_tc_tiling_on_sc=False, needs_layout_passes=False),
    )
    def k1(tab_hbm, out_hbm, vin_a, vin_b, vout, si_a, si_b):
        wid = lax.axis_index("s") * NC + lax.axis_index("c")
        iota = lax.iota(jnp.int32, L)
        row_base = (iota // 8) * VR2 + (iota % 8)
        z, o = jnp.int32(0), jnp.int32(1)

        def issue_in(s, vin, sem):
            tcb8 = (wid * cols_w + s * CW) // 128 * 8
            pltpu.async_copy(tab_hbm.at[z].at[pl.ds(tcb8, 128)],
                             vin.at[pl.ds(0, 128), pl.ds(0, 128)], sem)
            pltpu.async_copy(tab_hbm.at[o].at[pl.ds(tcb8, 128)],
                             vin.at[pl.ds(VR2, 128), pl.ds(0, 128)], sem)

        def wait_in(vin, sem):
            pltpu.make_async_copy(
                tab_hbm.at[z].at[pl.ds(0, 128)],
                vin.at[pl.ds(0, 128), pl.ds(0, 128)], sem).wait()
            pltpu.make_async_copy(
                tab_hbm.at[z].at[pl.ds(0, 128)],
                vin.at[pl.ds(VR2, 128), pl.ds(0, 128)], sem).wait()

        def transpose(vin, vout):
            def tr_body(g, carry2):
                rvec = row_base + (g // 8) * 8
                c0 = (g % 8) * L
                for u in range(UNR):
                    vec = plsc.load_gather(
                        vin, [rvec, jnp.full((L,), c0 + u, jnp.int32)])
                    vout[2 * g + u // 8, pl.ds((u % 8) * L, L)] = vec
                return carry2

            lax.fori_loop(jnp.int32(0), jnp.int32(CW // UNR), tr_body, 0)

        issue_in(jnp.int32(0), vin_a, si_a)
        issue_in(jnp.int32(1), vin_b, si_b)

        def pair_body(t, carry):
            for par, vin, si in ((0, vin_a, si_a), (1, vin_b, si_b)):
                s = 2 * t + par
                wait_in(vin, si)
                transpose(vin, vout)

                @pl.when(t < (n_sub // 2) - 1)
                def _():
                    issue_in(s + 2, vin, si)

                orow0 = (wid * cols_w + s * CW) * D // 128
                pltpu.sync_copy(vout, out_hbm.at[pl.ds(orow0, OUT_CH)])
            return carry

        lax.fori_loop(jnp.int32(0), jnp.int32(n_sub // 2), pair_body, 0)

    return k1


@functools.cache
def _make_k2(B, V, D, NC, NS):
    NW = NC * NS
    b_w = B // NW
    CH = 2048
    n_sub = b_w // CH
    TC_CH = CH // 128
    VR = TC_CH * 8
    UNR = 8

    @functools.partial(
        pl.kernel,
        out_type=jax.ShapeDtypeStruct((D // 8, B // 128 * 8, 128),
                                      jnp.float32),
        mesh=_mesh(NC, NS),
        scratch_types=[
            pltpu.VMEM((CH,), jnp.int32),
            pltpu.VMEM((CH,), jnp.int32),
            pltpu.VMEM((CH,), jnp.int32),
            pltpu.VMEM((CH,), jnp.int32),
            pltpu.VMEM((CH, D), jnp.float32),
            pltpu.VMEM((2 * VR, 129), jnp.float32),
            pltpu.SemaphoreType.DMA,
        ],
        compiler_params=pltpu.CompilerParams(
            use_tc_tiling_on_sc=False, needs_layout_passes=False),
    )
    def k2(i0_hbm, i1_hbm, i2_hbm, table_hbm, out_hbm,
           i0_v, i1_v, i2_v, h_v, rows_v, vout, sem):
        wid = lax.axis_index("s") * NC + lax.axis_index("c")
        base_w = wid * b_w
        iota = lax.iota(jnp.int32, L)
        row_const = (iota // 8) * VR + (iota % 8)

        def sub_body(s, carry):
            base = base_w + s * CH
            pltpu.sync_copy(i0_hbm.at[pl.ds(base, CH)], i0_v)
            pltpu.sync_copy(i1_hbm.at[pl.ds(base, CH)], i1_v)
            pltpu.sync_copy(i2_hbm.at[pl.ds(base, CH)], i2_v)

            def hash_body(j, carry2):
                a = i0_v[pl.ds(j * L, L)]
                b = i1_v[pl.ds(j * L, L)]
                c = i2_v[pl.ds(j * L, L)]
                h_v[pl.ds(j * L, L)] = (a ^ (b * P1) ^ (c * P2)) & MASK
                return carry2

            lax.fori_loop(jnp.int32(0), jnp.int32(CH // L), hash_body, 0)
            pltpu.async_copy(table_hbm.at[h_v], rows_v, sem).wait()

            def tr_body(g, carry2):
                b0 = g * UNR
                rvec = row_const + (b0 // 128) * 8
                c0 = b0 % 128
                for u in range(UNR):
                    v = rows_v[b0 + u, pl.ds(0, L)]
                    plsc.store_scatter(
                        vout, [rvec, jnp.full((L,), c0 + u, jnp.int32)], v)
                return carry2

            lax.fori_loop(jnp.int32(0), jnp.int32(CH // UNR), tr_body, 0)

            tcr0 = base // 128 * 8
            z, o = jnp.int32(0), jnp.int32(1)
            pltpu.sync_copy(vout.at[pl.ds(0, VR), pl.ds(0, 128)],
                            out_hbm.at[z].at[pl.ds(tcr0, VR)])
            pltpu.sync_copy(vout.at[pl.ds(VR, VR), pl.ds(0, 128)],
                            out_hbm.at[o].at[pl.ds(tcr0, VR)])
            return carry

        lax.fori_loop(jnp.int32(0), jnp.int32(n_sub), sub_body, 0)

    return k2


def kernel(index, hash_table):
    B = index.shape[0]
    V, D = hash_table.shape
    try:
        info = plsc.get_sparse_core_info()
        NC, NS = info.num_cores, info.num_subcores
    except Exception:
        NC, NS = 2, 16
    idx32 = index.astype(jnp.int32)
    tab_bytes = (hash_table.reshape(V // 128, 128, D // 8, 8)
                 .transpose(2, 0, 3, 1).reshape(D // 8, V // 128 * 8, 128))
    table_lin = _make_k1(V, D, NC, NS)(tab_bytes)
    out3d = _make_k2(B, V, D, NC, NS)(
        idx32[:, 0], idx32[:, 1], idx32[:, 2],
        table_lin.reshape(V, D))
    return (out3d.reshape(D // 8, B // 128, 8, 128)
            .transpose(1, 3, 0, 2).reshape(B, D))
